# baseline (speedup 1.0000x reference)
import math

import numpy as np
import jax
import jax.numpy as jnp
from jax import lax
from jax.experimental import pallas as pl
from jax.experimental.pallas import tpu as pltpu

_H = 6
_WS = 8
_N = _WS * _WS
_P = _N * _N
_T = (2 * _WS - 1) ** 2


def _build_tables():
    ch = np.arange(-(_WS - 1), _WS, dtype=np.float32)
    t = np.stack(np.meshgrid(ch, ch, indexing="ij"), axis=-1)
    t /= float(_WS - 1)
    t *= 8.0
    t = np.sign(t) * np.log2(np.abs(t) + 1.0) / np.log2(8.0)
    coords_t = t.reshape(_T, 2).T.copy()

    c = np.arange(_WS)
    grid = np.stack(np.meshgrid(c, c, indexing="ij")).reshape(2, -1)
    rel = (grid[:, :, None] - grid[:, None, :]).transpose(1, 2, 0)
    rel = rel.astype(np.int64)
    rel[:, :, 0] += _WS - 1
    rel[:, :, 1] += _WS - 1
    rel[:, :, 0] *= 2 * _WS - 1
    idx = rel.sum(-1).reshape(-1)

    onehot_t = np.zeros((_T, _P), dtype=np.float32)
    onehot_t[idx, np.arange(_P)] = 1.0
    return coords_t, onehot_t


_TT_NP, _OT_NP = _build_tables()


def _bias_kernel(ls_ref, w1_ref, b1_ref, w2_ref, tt_ref, ot_ref,
                 bias_ref, scale_ref):
    h = lax.dot_general(w1_ref[...], tt_ref[...], (((0,), (0,)), ((), ())),
                        preferred_element_type=jnp.float32)
    h = jnp.maximum(h + b1_ref[...], 0.0)
    bt = lax.dot_general(w2_ref[...], h, (((0,), (0,)), ((), ())),
                         preferred_element_type=jnp.float32)
    bias = jnp.dot(bt, ot_ref[...], preferred_element_type=jnp.float32)
    bias_ref[...] = 16.0 * jax.nn.sigmoid(bias)
    scale_ref[...] = jnp.exp(jnp.minimum(ls_ref[...], math.log(100.0)))


def _apply_kernel(attn_ref, scale_ref, bias_ref, out_ref):
    s = scale_ref[...][None, :, :, None]
    out_ref[...] = attn_ref[...] * s + bias_ref[...][None]


def kernel(attn, x_size, logit_scale, w1, b1, w2):
    del x_size
    B = attn.shape[0]

    tt = jnp.asarray(_TT_NP)
    ot = jnp.asarray(_OT_NP)
    ls2 = logit_scale.reshape(_H, 1)
    b1c = b1.reshape(-1, 1)

    bias2, scale = pl.pallas_call(
        _bias_kernel,
        out_shape=(
            jax.ShapeDtypeStruct((_H, _P), jnp.float32),
            jax.ShapeDtypeStruct((_H, 1), jnp.float32),
        ),
    )(ls2, w1, b1c, w2, tt, ot)
    bias = bias2.reshape(_H, _N, _N)

    BB = 8
    out = pl.pallas_call(
        _apply_kernel,
        grid=(B // BB,),
        in_specs=[
            pl.BlockSpec((BB, _H, _N, _N), lambda i: (i, 0, 0, 0)),
            pl.BlockSpec((_H, 1), lambda i: (0, 0)),
            pl.BlockSpec((_H, _N, _N), lambda i: (0, 0, 0)),
        ],
        out_specs=pl.BlockSpec((BB, _H, _N, _N), lambda i: (i, 0, 0, 0)),
        out_shape=jax.ShapeDtypeStruct((B, _H, _N, _N), jnp.float32),
        compiler_params=pltpu.CompilerParams(
            dimension_semantics=("arbitrary",),
        ),
    )(attn, scale, bias)
    return out


# speedup vs baseline: 6.3103x; 6.3103x over previous
import math

import numpy as np
import jax
import jax.numpy as jnp
from jax.experimental import pallas as pl
from jax.experimental.pallas import tpu as pltpu

_H = 6
_WS = 8
_N = _WS * _WS
_P = _N * _N
_T = (2 * _WS - 1) ** 2
_IB = 16
_LB = 128


def _build_tables():
    ch = np.arange(-(_WS - 1), _WS, dtype=np.float32)
    t = np.stack(np.meshgrid(ch, ch, indexing="ij"), axis=-1)
    t /= float(_WS - 1)
    t *= 8.0
    t = np.sign(t) * np.log2(np.abs(t) + 1.0) / np.log2(8.0)
    coords = t.reshape(_T, 2)

    c = np.arange(_WS)
    grid = np.stack(np.meshgrid(c, c, indexing="ij")).reshape(2, -1)
    rel = (grid[:, :, None] - grid[:, None, :]).transpose(1, 2, 0)
    rel = rel.astype(np.int64)
    rel[:, :, 0] += _WS - 1
    rel[:, :, 1] += _WS - 1
    rel[:, :, 0] *= 2 * _WS - 1
    idx = rel.sum(-1).reshape(-1)

    onehot = np.zeros((_P, _T), dtype=np.float32)
    onehot[np.arange(_P), idx] = 1.0
    return coords, onehot


_TC_NP, _OC_NP = _build_tables()


def _fused_kernel(ls_ref, w1_ref, b1_ref, w2_ref, tc_ref, oc_ref, attn_ref,
                  out_ref, bias_vmem, scale_vmem):
    i = pl.program_id(0)
    j = pl.program_id(1)

    @pl.when(jnp.logical_and(i == 0, j == 0))
    def _prologue():
        h = jnp.dot(tc_ref[...], w1_ref[...],
                    preferred_element_type=jnp.float32)
        h = jnp.maximum(h + b1_ref[...], 0.0)
        tbl = jnp.dot(h, w2_ref[...],
                      preferred_element_type=jnp.float32)
        tbl = 16.0 * jax.nn.sigmoid(tbl)
        bvt = jnp.dot(oc_ref[...], tbl,
                      preferred_element_type=jnp.float32)
        for hd in range(_H):
            col = bvt[:, hd:hd + 1].reshape(_N, _N, 1)
            bias_vmem[hd, :, :, :] = jnp.broadcast_to(col, (_N, _N, _LB))
        scale_vmem[...] = jnp.exp(jnp.minimum(ls_ref[...], math.log(100.0)))

    s = scale_vmem[...][:, :, None, None]
    b = bias_vmem[:, pl.ds(i * _IB, _IB), :, :]
    out_ref[...] = attn_ref[...] * s + b


def kernel(attn, x_size, logit_scale, w1, b1, w2):
    del x_size
    B = attn.shape[0]
    attn_t = jnp.transpose(attn, (1, 2, 3, 0))

    tc = jnp.asarray(_TC_NP)
    oc = jnp.asarray(_OC_NP)
    ls2 = logit_scale.reshape(_H, 1)
    b1r = b1.reshape(1, -1)

    out_t = pl.pallas_call(
        _fused_kernel,
        grid=(_N // _IB, B // _LB),
        in_specs=[
            pl.BlockSpec((_H, 1), lambda i, j: (0, 0)),
            pl.BlockSpec((2, 512), lambda i, j: (0, 0)),
            pl.BlockSpec((1, 512), lambda i, j: (0, 0)),
            pl.BlockSpec((512, _H), lambda i, j: (0, 0)),
            pl.BlockSpec((_T, 2), lambda i, j: (0, 0)),
            pl.BlockSpec((_P, _T), lambda i, j: (0, 0)),
            pl.BlockSpec((_H, _IB, _N, _LB), lambda i, j: (0, i, 0, j)),
        ],
        out_specs=pl.BlockSpec((_H, _IB, _N, _LB), lambda i, j: (0, i, 0, j)),
        out_shape=jax.ShapeDtypeStruct((_H, _N, _N, B), jnp.float32),
        scratch_shapes=[
            pltpu.VMEM((_H, _N, _N, _LB), jnp.float32),
            pltpu.VMEM((_H, 1), jnp.float32),
        ],
        compiler_params=pltpu.CompilerParams(
            dimension_semantics=("arbitrary", "arbitrary"),
        ),
    )(ls2, w1, b1r, w2, tc, oc, attn_t)
    return jnp.transpose(out_t, (3, 0, 1, 2))


# speedup vs baseline: 6.3721x; 1.0098x over previous
import math

import numpy as np
import jax
import jax.numpy as jnp
from jax.experimental import pallas as pl
from jax.experimental.pallas import tpu as pltpu

_H = 6
_WS = 8
_N = _WS * _WS
_P = _N * _N
_T = (2 * _WS - 1) ** 2
_RB = 1024
_LB = 128


def _build_tables():
    ch = np.arange(-(_WS - 1), _WS, dtype=np.float32)
    t = np.stack(np.meshgrid(ch, ch, indexing="ij"), axis=-1)
    t /= float(_WS - 1)
    t *= 8.0
    t = np.sign(t) * np.log2(np.abs(t) + 1.0) / np.log2(8.0)
    coords = t.reshape(_T, 2)

    c = np.arange(_WS)
    grid = np.stack(np.meshgrid(c, c, indexing="ij")).reshape(2, -1)
    rel = (grid[:, :, None] - grid[:, None, :]).transpose(1, 2, 0)
    rel = rel.astype(np.int64)
    rel[:, :, 0] += _WS - 1
    rel[:, :, 1] += _WS - 1
    rel[:, :, 0] *= 2 * _WS - 1
    idx = rel.sum(-1).reshape(-1)

    onehot = np.zeros((_P, _T), dtype=np.float32)
    onehot[np.arange(_P), idx] = 1.0
    return coords, onehot


_TC_NP, _OC_NP = _build_tables()


def _fused_kernel(ls_ref, w1_ref, b1_ref, w2_ref, tc_ref, oc_ref, attn_ref,
                  out_ref, bias_vmem, scale_vmem):
    i = pl.program_id(0)
    j = pl.program_id(1)

    @pl.when(jnp.logical_and(i == 0, j == 0))
    def _prologue():
        h = jnp.dot(tc_ref[...], w1_ref[...],
                    preferred_element_type=jnp.float32)
        h = jnp.maximum(h + b1_ref[...], 0.0)
        tbl = jnp.dot(h, w2_ref[...],
                      preferred_element_type=jnp.float32)
        tbl = 16.0 * jax.nn.sigmoid(tbl)
        tbl_hi = tbl.astype(jnp.bfloat16)
        tbl_lo = (tbl - tbl_hi.astype(jnp.float32)).astype(jnp.bfloat16)
        oc = oc_ref[...]
        bvt = (jnp.dot(oc, tbl_hi, preferred_element_type=jnp.float32) +
               jnp.dot(oc, tbl_lo, preferred_element_type=jnp.float32))
        for hd in range(_H):
            bias_vmem[hd, :, :] = jnp.broadcast_to(bvt[:, hd:hd + 1],
                                                   (_P, _LB))
        scale_vmem[...] = jnp.exp(jnp.minimum(ls_ref[...], math.log(100.0)))

    s = scale_vmem[...][:, :, None]
    b = bias_vmem[:, pl.ds(i * _RB, _RB), :]
    out_ref[...] = attn_ref[...] * s + b


def kernel(attn, x_size, logit_scale, w1, b1, w2):
    del x_size
    B = attn.shape[0]
    attn_t = jnp.transpose(attn, (1, 2, 3, 0)).reshape(_H, _P, B)

    tc = jnp.asarray(_TC_NP)
    oc = jnp.asarray(_OC_NP, dtype=jnp.bfloat16)
    ls2 = logit_scale.reshape(_H, 1)
    b1r = b1.reshape(1, -1)

    out_t = pl.pallas_call(
        _fused_kernel,
        grid=(_P // _RB, B // _LB),
        in_specs=[
            pl.BlockSpec((_H, 1), lambda i, j: (0, 0)),
            pl.BlockSpec((2, 512), lambda i, j: (0, 0)),
            pl.BlockSpec((1, 512), lambda i, j: (0, 0)),
            pl.BlockSpec((512, _H), lambda i, j: (0, 0)),
            pl.BlockSpec((_T, 2), lambda i, j: (0, 0)),
            pl.BlockSpec((_P, _T), lambda i, j: (0, 0)),
            pl.BlockSpec((_H, _RB, _LB), lambda i, j: (0, i, j)),
        ],
        out_specs=pl.BlockSpec((_H, _RB, _LB), lambda i, j: (0, i, j)),
        out_shape=jax.ShapeDtypeStruct((_H, _P, B), jnp.float32),
        scratch_shapes=[
            pltpu.VMEM((_H, _P, _LB), jnp.float32),
            pltpu.VMEM((_H, 1), jnp.float32),
        ],
        compiler_params=pltpu.CompilerParams(
            dimension_semantics=("arbitrary", "arbitrary"),
        ),
    )(ls2, w1, b1r, w2, tc, oc, attn_t)
    return jnp.transpose(out_t.reshape(_H, _N, _N, B), (3, 0, 1, 2))


# speedup vs baseline: 6.6600x; 1.0452x over previous
import math

import numpy as np
import jax
import jax.numpy as jnp
from jax.experimental import pallas as pl
from jax.experimental.pallas import tpu as pltpu

_H = 6
_WS = 8
_N = _WS * _WS
_P = _N * _N
_T = (2 * _WS - 1) ** 2
_RB = 2048
_LB = 128


def _build_tables():
    ch = np.arange(-(_WS - 1), _WS, dtype=np.float32)
    t = np.stack(np.meshgrid(ch, ch, indexing="ij"), axis=-1)
    t /= float(_WS - 1)
    t *= 8.0
    t = np.sign(t) * np.log2(np.abs(t) + 1.0) / np.log2(8.0)
    coords = t.reshape(_T, 2)

    c = np.arange(_WS)
    grid = np.stack(np.meshgrid(c, c, indexing="ij")).reshape(2, -1)
    rel = (grid[:, :, None] - grid[:, None, :]).transpose(1, 2, 0)
    rel = rel.astype(np.int64)
    rel[:, :, 0] += _WS - 1
    rel[:, :, 1] += _WS - 1
    rel[:, :, 0] *= 2 * _WS - 1
    idx = rel.sum(-1).reshape(-1)

    onehot = np.zeros((_P, _T), dtype=np.float32)
    onehot[np.arange(_P), idx] = 1.0
    return coords, onehot


_TC_NP, _OC_NP = _build_tables()


def _fused_kernel(ls_ref, w1_ref, b1_ref, w2_ref, tc_ref, oc_ref, attn_ref,
                  out_ref, bias_vmem, scale_vmem):
    i = pl.program_id(0)
    j = pl.program_id(1)

    @pl.when(jnp.logical_and(i == 0, j == 0))
    def _prologue():
        h = jnp.dot(tc_ref[...], w1_ref[...],
                    preferred_element_type=jnp.float32)
        h = jnp.maximum(h + b1_ref[...], 0.0)
        tbl = jnp.dot(h, w2_ref[...],
                      preferred_element_type=jnp.float32)
        tbl = 16.0 * jax.nn.sigmoid(tbl)
        tbl_hi = tbl.astype(jnp.bfloat16)
        tbl_lo = (tbl - tbl_hi.astype(jnp.float32)).astype(jnp.bfloat16)
        oc = oc_ref[...]
        bvt = (jnp.dot(oc, tbl_hi, preferred_element_type=jnp.float32) +
               jnp.dot(oc, tbl_lo, preferred_element_type=jnp.float32))
        for hd in range(_H):
            bias_vmem[hd, :, :] = jnp.broadcast_to(bvt[:, hd:hd + 1],
                                                   (_P, _LB))
        scale_vmem[...] = jnp.exp(jnp.minimum(ls_ref[...], math.log(100.0)))

    s = scale_vmem[...][:, :, None]
    b = bias_vmem[:, pl.ds(i * _RB, _RB), :]
    out_ref[...] = attn_ref[...] * s + b


def kernel(attn, x_size, logit_scale, w1, b1, w2):
    del x_size
    B = attn.shape[0]
    attn_t = jnp.transpose(attn, (1, 2, 3, 0)).reshape(_H, _P, B)

    tc = jnp.asarray(_TC_NP)
    oc = jnp.asarray(_OC_NP, dtype=jnp.bfloat16)
    ls2 = logit_scale.reshape(_H, 1)
    b1r = b1.reshape(1, -1)

    out_t = pl.pallas_call(
        _fused_kernel,
        grid=(_P // _RB, B // _LB),
        in_specs=[
            pl.BlockSpec((_H, 1), lambda i, j: (0, 0)),
            pl.BlockSpec((2, 512), lambda i, j: (0, 0)),
            pl.BlockSpec((1, 512), lambda i, j: (0, 0)),
            pl.BlockSpec((512, _H), lambda i, j: (0, 0)),
            pl.BlockSpec((_T, 2), lambda i, j: (0, 0)),
            pl.BlockSpec((_P, _T), lambda i, j: (0, 0)),
            pl.BlockSpec((_H, _RB, _LB), lambda i, j: (0, i, j)),
        ],
        out_specs=pl.BlockSpec((_H, _RB, _LB), lambda i, j: (0, i, j)),
        out_shape=jax.ShapeDtypeStruct((_H, _P, B), jnp.float32),
        scratch_shapes=[
            pltpu.VMEM((_H, _P, _LB), jnp.float32),
            pltpu.VMEM((_H, 1), jnp.float32),
        ],
        compiler_params=pltpu.CompilerParams(
            dimension_semantics=("arbitrary", "arbitrary"),
            vmem_limit_bytes=56 * 1024 * 1024,
        ),
    )(ls2, w1, b1r, w2, tc, oc, attn_t)
    return jnp.transpose(out_t.reshape(_H, _N, _N, B), (3, 0, 1, 2))


# speedup vs baseline: 6.8349x; 1.0263x over previous
import math

import numpy as np
import jax
import jax.numpy as jnp
from jax.experimental import pallas as pl
from jax.experimental.pallas import tpu as pltpu

_H = 6
_WS = 8
_N = _WS * _WS
_P = _N * _N
_T = (2 * _WS - 1) ** 2
_RB = 1024
_LB = 512
_LS = 128


def _build_tables():
    ch = np.arange(-(_WS - 1), _WS, dtype=np.float32)
    t = np.stack(np.meshgrid(ch, ch, indexing="ij"), axis=-1)
    t /= float(_WS - 1)
    t *= 8.0
    t = np.sign(t) * np.log2(np.abs(t) + 1.0) / np.log2(8.0)
    coords = t.reshape(_T, 2)

    c = np.arange(_WS)
    grid = np.stack(np.meshgrid(c, c, indexing="ij")).reshape(2, -1)
    rel = (grid[:, :, None] - grid[:, None, :]).transpose(1, 2, 0)
    rel = rel.astype(np.int64)
    rel[:, :, 0] += _WS - 1
    rel[:, :, 1] += _WS - 1
    rel[:, :, 0] *= 2 * _WS - 1
    idx = rel.sum(-1).reshape(-1)

    onehot = np.zeros((_P, _T), dtype=np.float32)
    onehot[np.arange(_P), idx] = 1.0
    return coords, onehot


_TC_NP, _OC_NP = _build_tables()


def _fused_kernel(ls_ref, w1_ref, b1_ref, w2_ref, tc_ref, oc_ref, attn_ref,
                  out_ref, bias_vmem, scale_vmem):
    j = pl.program_id(1)

    @pl.when(j == 0)
    def _prologue():
        h = jnp.dot(tc_ref[...], w1_ref[...],
                    preferred_element_type=jnp.float32)
        h = jnp.maximum(h + b1_ref[...], 0.0)
        tbl = jnp.dot(h, w2_ref[...],
                      preferred_element_type=jnp.float32)
        tbl = 16.0 * jax.nn.sigmoid(tbl)
        tbl_hi = tbl.astype(jnp.bfloat16)
        tbl_lo = (tbl - tbl_hi.astype(jnp.float32)).astype(jnp.bfloat16)
        oc = oc_ref[...]
        bvt = (jnp.dot(oc, tbl_hi, preferred_element_type=jnp.float32) +
               jnp.dot(oc, tbl_lo, preferred_element_type=jnp.float32))
        for hd in range(_H):
            bias_vmem[hd, :, :] = jnp.broadcast_to(bvt[:, hd:hd + 1],
                                                   (_RB, _LS))
        scale_vmem[...] = jnp.exp(jnp.minimum(ls_ref[...], math.log(100.0)))

    s = scale_vmem[...][:, :, None]
    bs = bias_vmem[...]
    for half in range(_LB // _LS):
        sl = slice(half * _LS, (half + 1) * _LS)
        out_ref[:, :, sl] = attn_ref[:, :, sl] * s + bs


def kernel(attn, x_size, logit_scale, w1, b1, w2):
    del x_size
    B = attn.shape[0]
    attn_t = jnp.transpose(attn, (1, 2, 3, 0)).reshape(_H, _P, B)

    tc = jnp.asarray(_TC_NP)
    oc = jnp.asarray(_OC_NP, dtype=jnp.bfloat16)
    ls2 = logit_scale.reshape(_H, 1)
    b1r = b1.reshape(1, -1)

    out_t = pl.pallas_call(
        _fused_kernel,
        grid=(_P // _RB, B // _LB),
        in_specs=[
            pl.BlockSpec((_H, 1), lambda i, j: (0, 0)),
            pl.BlockSpec((2, 512), lambda i, j: (0, 0)),
            pl.BlockSpec((1, 512), lambda i, j: (0, 0)),
            pl.BlockSpec((512, _H), lambda i, j: (0, 0)),
            pl.BlockSpec((_T, 2), lambda i, j: (0, 0)),
            pl.BlockSpec((_RB, _T), lambda i, j: (i, 0)),
            pl.BlockSpec((_H, _RB, _LB), lambda i, j: (0, i, j)),
        ],
        out_specs=pl.BlockSpec((_H, _RB, _LB), lambda i, j: (0, i, j)),
        out_shape=jax.ShapeDtypeStruct((_H, _P, B), jnp.float32),
        scratch_shapes=[
            pltpu.VMEM((_H, _RB, _LS), jnp.float32),
            pltpu.VMEM((_H, 1), jnp.float32),
        ],
        compiler_params=pltpu.CompilerParams(
            dimension_semantics=("arbitrary", "arbitrary"),
            vmem_limit_bytes=60 * 1024 * 1024,
        ),
    )(ls2, w1, b1r, w2, tc, oc, attn_t)
    return jnp.transpose(out_t.reshape(_H, _N, _N, B), (3, 0, 1, 2))


# speedup vs baseline: 7.1237x; 1.0423x over previous
import math

import numpy as np
import jax
import jax.numpy as jnp
from jax.experimental import pallas as pl
from jax.experimental.pallas import tpu as pltpu

_H = 6
_WS = 8
_N = _WS * _WS
_P = _N * _N
_T = (2 * _WS - 1) ** 2
_RB = 1024
_LB = 512
_LS = 128


def _build_tables():
    ch = np.arange(-(_WS - 1), _WS, dtype=np.float32)
    t = np.stack(np.meshgrid(ch, ch, indexing="ij"), axis=-1)
    t /= float(_WS - 1)
    t *= 8.0
    t = np.sign(t) * np.log2(np.abs(t) + 1.0) / np.log2(8.0)
    coords = t.reshape(_T, 2)

    c = np.arange(_WS)
    grid = np.stack(np.meshgrid(c, c, indexing="ij")).reshape(2, -1)
    rel = (grid[:, :, None] - grid[:, None, :]).transpose(1, 2, 0)
    rel = rel.astype(np.int64)
    rel[:, :, 0] += _WS - 1
    rel[:, :, 1] += _WS - 1
    rel[:, :, 0] *= 2 * _WS - 1
    idx = rel.sum(-1).reshape(-1)

    onehot = np.zeros((_P, _T), dtype=np.float32)
    onehot[np.arange(_P), idx] = 1.0
    return coords, onehot


_TC_NP, _OC_NP = _build_tables()


def _fused_kernel(ls_ref, w1_ref, b1_ref, w2_ref, tc_ref, oc_ref, attn_ref,
                  out_ref, bias_vmem, scale_vmem):
    j = pl.program_id(1)

    @pl.when(j == 0)
    def _prologue():
        h = jnp.dot(tc_ref[...], w1_ref[...],
                    preferred_element_type=jnp.float32)
        h = jnp.maximum(h + b1_ref[...], 0.0)
        tbl = jax.lax.dot_general(h, w2_ref[...],
                                  (((1,), (1,)), ((), ())),
                                  preferred_element_type=jnp.float32)
        tbl = 16.0 * jax.nn.sigmoid(tbl)
        tbl_hi = tbl.astype(jnp.bfloat16)
        tbl_lo = (tbl - tbl_hi.astype(jnp.float32)).astype(jnp.bfloat16)
        oc = oc_ref[...]
        bvt = (jnp.dot(oc, tbl_hi, preferred_element_type=jnp.float32) +
               jnp.dot(oc, tbl_lo, preferred_element_type=jnp.float32))
        for hd in range(_H):
            bias_vmem[hd, :, :] = jnp.broadcast_to(bvt[:, hd:hd + 1],
                                                   (_RB, _LS))
        sc = jnp.exp(jnp.minimum(ls_ref[...], math.log(100.0)))
        scale_vmem[...] = jnp.transpose(sc, (1, 0))

    s = scale_vmem[...][:, :, None]
    bs = bias_vmem[...]
    for half in range(_LB // _LS):
        sl = slice(half * _LS, (half + 1) * _LS)
        out_ref[:, :, sl] = attn_ref[:, :, sl] * s + bs


def kernel(attn, x_size, logit_scale, w1, b1, w2):
    del x_size
    B = attn.shape[0]
    attn_t = jnp.transpose(attn, (1, 2, 3, 0)).reshape(_H, _P, B)

    tc = jnp.asarray(_TC_NP)
    oc = jnp.asarray(_OC_NP, dtype=jnp.bfloat16)
    ls2 = logit_scale.reshape(1, _H)
    b1r = b1.reshape(1, -1)

    out_t = pl.pallas_call(
        _fused_kernel,
        grid=(_P // _RB, B // _LB),
        in_specs=[
            pl.BlockSpec((1, _H), lambda i, j: (0, 0)),
            pl.BlockSpec((2, 512), lambda i, j: (0, 0)),
            pl.BlockSpec((1, 512), lambda i, j: (0, 0)),
            pl.BlockSpec((_H, 512), lambda i, j: (0, 0)),
            pl.BlockSpec((_T, 2), lambda i, j: (0, 0)),
            pl.BlockSpec((_RB, _T), lambda i, j: (i, 0)),
            pl.BlockSpec((_H, _RB, _LB), lambda i, j: (0, i, j)),
        ],
        out_specs=pl.BlockSpec((_H, _RB, _LB), lambda i, j: (0, i, j)),
        out_shape=jax.ShapeDtypeStruct((_H, _P, B), jnp.float32),
        scratch_shapes=[
            pltpu.VMEM((_H, _RB, _LS), jnp.float32),
            pltpu.VMEM((_H, 1), jnp.float32),
        ],
        compiler_params=pltpu.CompilerParams(
            dimension_semantics=("arbitrary", "arbitrary"),
            vmem_limit_bytes=60 * 1024 * 1024,
        ),
    )(ls2, w1, b1r, w2.T, tc, oc, attn_t)
    return jnp.transpose(out_t.reshape(_H, _N, _N, B), (3, 0, 1, 2))


# speedup vs baseline: 7.6112x; 1.0684x over previous
import math

import numpy as np
import jax
import jax.numpy as jnp
from jax.experimental import pallas as pl
from jax.experimental.pallas import tpu as pltpu

_H = 6
_WS = 8
_N = _WS * _WS
_P = _N * _N
_T = (2 * _WS - 1) ** 2
_RB = 512
_LS = 128


def _build_tables():
    ch = np.arange(-(_WS - 1), _WS, dtype=np.float32)
    t = np.stack(np.meshgrid(ch, ch, indexing="ij"), axis=-1)
    t /= float(_WS - 1)
    t *= 8.0
    t = np.sign(t) * np.log2(np.abs(t) + 1.0) / np.log2(8.0)
    coords = t.reshape(_T, 2)

    c = np.arange(_WS)
    grid = np.stack(np.meshgrid(c, c, indexing="ij")).reshape(2, -1)
    rel = (grid[:, :, None] - grid[:, None, :]).transpose(1, 2, 0)
    rel = rel.astype(np.int64)
    rel[:, :, 0] += _WS - 1
    rel[:, :, 1] += _WS - 1
    rel[:, :, 0] *= 2 * _WS - 1
    idx = rel.sum(-1).reshape(-1)

    onehot = np.zeros((_P, _T), dtype=np.float32)
    onehot[np.arange(_P), idx] = 1.0
    return coords, onehot


_TC_NP, _OC_NP = _build_tables()


def _fused_kernel(ls_ref, w1_ref, b1_ref, w2_ref, tc_ref, oc_ref, attn_ref,
                  out_ref, bvt_vmem, scale_vmem):
    i = pl.program_id(0)

    @pl.when(i == 0)
    def _prologue():
        h = jnp.dot(tc_ref[...], w1_ref[...],
                    preferred_element_type=jnp.float32)
        h = jnp.maximum(h + b1_ref[...], 0.0)
        tbl = jax.lax.dot_general(h, w2_ref[...],
                                  (((1,), (1,)), ((), ())),
                                  preferred_element_type=jnp.float32)
        tbl = 16.0 * jax.nn.sigmoid(tbl)
        tbl_hi = tbl.astype(jnp.bfloat16)
        tbl_lo = (tbl - tbl_hi.astype(jnp.float32)).astype(jnp.bfloat16)
        oc = oc_ref[...]
        bvt_vmem[...] = (
            jnp.dot(oc, tbl_hi, preferred_element_type=jnp.float32) +
            jnp.dot(oc, tbl_lo, preferred_element_type=jnp.float32))
        sc = jnp.exp(jnp.minimum(ls_ref[...], math.log(100.0)))
        scale_vmem[...] = jnp.transpose(sc, (1, 0))

    nlt = attn_ref.shape[2] // _LS
    for hd in range(_H):
        bh = jnp.broadcast_to(
            bvt_vmem[pl.ds(i * _RB, _RB), hd:hd + 1], (_RB, _LS))
        sh = scale_vmem[hd, 0]
        for lt in range(nlt):
            sl = slice(lt * _LS, (lt + 1) * _LS)
            out_ref[hd, :, sl] = attn_ref[hd, :, sl] * sh + bh


def kernel(attn, x_size, logit_scale, w1, b1, w2):
    del x_size
    B = attn.shape[0]
    attn_t = jnp.transpose(attn, (1, 2, 3, 0)).reshape(_H, _P, B)

    tc = jnp.asarray(_TC_NP)
    oc = jnp.asarray(_OC_NP, dtype=jnp.bfloat16)
    ls2 = logit_scale.reshape(1, _H)
    b1r = b1.reshape(1, -1)

    out_t = pl.pallas_call(
        _fused_kernel,
        grid=(_P // _RB,),
        in_specs=[
            pl.BlockSpec((1, _H), lambda i: (0, 0)),
            pl.BlockSpec((2, 512), lambda i: (0, 0)),
            pl.BlockSpec((1, 512), lambda i: (0, 0)),
            pl.BlockSpec((_H, 512), lambda i: (0, 0)),
            pl.BlockSpec((_T, 2), lambda i: (0, 0)),
            pl.BlockSpec((_P, _T), lambda i: (0, 0)),
            pl.BlockSpec((_H, _RB, B), lambda i: (0, i, 0)),
        ],
        out_specs=pl.BlockSpec((_H, _RB, B), lambda i: (0, i, 0)),
        out_shape=jax.ShapeDtypeStruct((_H, _P, B), jnp.float32),
        scratch_shapes=[
            pltpu.VMEM((_P, _H), jnp.float32),
            pltpu.VMEM((_H, 1), jnp.float32),
        ],
        compiler_params=pltpu.CompilerParams(
            dimension_semantics=("arbitrary",),
            vmem_limit_bytes=60 * 1024 * 1024,
        ),
    )(ls2, w1, b1r, w2.T, tc, oc, attn_t)
    return jnp.transpose(out_t.reshape(_H, _N, _N, B), (3, 0, 1, 2))
